# baseline (speedup 1.0000x reference)
import jax
import jax.numpy as jnp
from jax import lax
from jax.experimental import pallas as pl
from jax.experimental.pallas import tpu as pltpu

_N = 3
_SEG = 128
_IN = 64
_OUT = 64


def _body(x_ref, w_ref, o_ref):
    xb = x_ref[...]
    t = xb.shape[0]
    seg_iota = lax.broadcasted_iota(jnp.int32, (t, _SEG), 1)
    acc = jnp.zeros((t, _OUT), jnp.float32)
    for j in range(_IN):
        xj = xb[:, j:j + 1]
        idf = jnp.floor((xj + 1.0) * (_SEG / 2.0))
        idf = jnp.clip(idf, 0.0, float(_SEG - 1))
        idi = idf.astype(jnp.int32)
        x_min = idf * (2.0 / _SEG) - 1.0
        u = (xj - x_min) * _SEG - 1.0
        b0 = 0.5 * u * (u - 1.0)
        b1 = 1.0 - u * u
        b2 = 0.5 * u * (u + 1.0)
        hot = jnp.where(seg_iota == idi, 1.0, 0.0).astype(jnp.bfloat16)
        r = lax.dot_general(
            hot, w_ref[j], (((1,), (0,)), ((), ())),
            preferred_element_type=jnp.float32)
        acc = acc + (b0 * r[:, :_OUT]
                     + b1 * r[:, _OUT:2 * _OUT]
                     + b2 * r[:, 2 * _OUT:])

    o_ref[...] = acc


@jax.jit
def kernel(x, w):
    batch = x.shape[0]
    t = 512
    wg = jnp.transpose(w.reshape(_OUT, _IN, _SEG, _N), (1, 2, 3, 0))
    wg = wg.reshape(_IN, _SEG, _N * _OUT).astype(jnp.bfloat16)
    grid = (batch // t,)
    return pl.pallas_call(
        _body,
        grid=grid,
        in_specs=[
            pl.BlockSpec((t, _IN), lambda i: (i, 0)),
            pl.BlockSpec((_IN, _SEG, _N * _OUT), lambda i: (0, 0, 0)),
        ],
        out_specs=pl.BlockSpec((t, _OUT), lambda i: (i, 0)),
        out_shape=jax.ShapeDtypeStruct((batch, _OUT), jnp.float32),
    )(x, wg)


# speedup vs baseline: 4.0391x; 4.0391x over previous
import jax
import jax.numpy as jnp
from jax import lax
from jax.experimental import pallas as pl
from jax.experimental.pallas import tpu as pltpu

_N = 3
_SEG = 128
_IN = 64
_OUT = 64
_JG = 32


def _body(xt_ref, w_ref, o_ref):
    g = pl.program_id(0)
    xb = xt_ref[...]
    t = xb.shape[1]
    seg_bf = lax.broadcasted_iota(jnp.int32, (_SEG, t), 0).astype(jnp.bfloat16)

    m_f = lax.broadcasted_iota(jnp.int32, (_N * _SEG, _SEG), 0).astype(jnp.float32)
    s3_f = lax.broadcasted_iota(jnp.int32, (_N * _SEG, _SEG), 1).astype(jnp.float32) * 3.0
    sel = [jnp.where(m_f == s3_f + float(k), 1.0, 0.0).astype(jnp.bfloat16)
           for k in range(_N)]

    acc = jnp.zeros((_OUT, t), jnp.float32)
    for k in range(_JG):
        j = k
        xj = xb[j:j + 1, :]
        idf = jnp.floor((xj + 1.0) * (_SEG / 2.0))
        idf = jnp.clip(idf, 0.0, float(_SEG - 1))
        x_min = idf * (2.0 / _SEG) - 1.0
        u = (xj - x_min) * _SEG - 1.0
        b0 = 0.5 * u * (u - 1.0)
        b1 = 1.0 - u * u
        b2 = 0.5 * u * (u + 1.0)
        id_bf = idf.astype(jnp.bfloat16)
        hot = jnp.where(seg_bf == id_bf, jnp.bfloat16(1), jnp.bfloat16(0))
        wj = w_ref[:, j, :].astype(jnp.bfloat16)
        wg = jnp.concatenate(
            [lax.dot_general(wj, sel[p], (((1,), (0,)), ((), ())),
                             preferred_element_type=jnp.float32
                             ).astype(jnp.bfloat16)
             for p in range(_N)], axis=0)
        r = lax.dot_general(
            wg, hot, (((1,), (0,)), ((), ())),
            preferred_element_type=jnp.float32)
        acc = acc + (b0 * r[:_OUT]
                     + b1 * r[_OUT:2 * _OUT]
                     + b2 * r[2 * _OUT:])

    @pl.when(g == 0)
    def _():
        o_ref[...] = jnp.zeros_like(o_ref)

    o_ref[...] += acc


@jax.jit
def kernel(x, w):
    batch = x.shape[0]
    xt = x.T
    grid = (_IN // _JG,)
    ot = pl.pallas_call(
        _body,
        grid=grid,
        in_specs=[
            pl.BlockSpec((_JG, batch), lambda g: (g, 0)),
            pl.BlockSpec((_OUT, _JG, _N * _SEG), lambda g: (0, g, 0)),
        ],
        out_specs=pl.BlockSpec((_OUT, batch), lambda g: (0, 0)),
        out_shape=jax.ShapeDtypeStruct((_OUT, batch), jnp.float32),
    )(xt, w)
    return ot.T
